# baseline (speedup 1.0000x reference)
import functools

import jax
import jax.numpy as jnp
from jax import lax
from jax.experimental import pallas as pl
from jax.experimental.pallas import tpu as pltpu
from jax.experimental.pallas import tpu_sc as plsc

B, S = 4, 2048
NTOK = B * S
D6 = 128
DOUT = 192
NSLOT = 6
TAB = 1024

_info = plsc.get_sparse_core_info()
NC = _info.num_cores
NS = _info.num_subcores
NW = NC * NS
TPW = NTOK // NW
CH = 64
NCH = TPW // CH
LANES = 16


def _tc_project_body(x_ref, y_ref, h_ref, w_ref, lw_ref, b_ref, out_ref):
    k = pl.program_id(0)
    emb = jnp.where(
        (k == 0) | (k == 2), x_ref[...],
        jnp.where((k == 1) | (k == 3), y_ref[...],
                  jnp.where(k == 4, h_ref[...], w_ref[...])))
    acc = lax.dot_general(emb, lw_ref[...], (((1,), (1,)), ((), ())),
                          preferred_element_type=jnp.float32)
    acc = acc + jnp.where(k == 0, b_ref[...], 0.0)
    cols = lax.broadcasted_iota(jnp.int32, (DOUT, DOUT), 1)
    rows = lax.broadcasted_iota(jnp.int32, (DOUT, DOUT), 0)
    src = (cols // 32) * 32 + (cols % 2) * 16 + (cols % 32) // 2
    perm = (rows == src).astype(jnp.float32)
    out_ref[...] = lax.dot_general(
        acc, perm, (((1,), (0,)), ((), ())),
        preferred_element_type=jnp.float32).astype(jnp.bfloat16)


def _tc_project(x_emb, y_emb, h_emb, w_emb, lin_w, bias):
    return pl.pallas_call(
        _tc_project_body,
        grid=(NSLOT,),
        in_specs=[
            pl.BlockSpec((TAB, D6), lambda k: (0, 0)),
            pl.BlockSpec((TAB, D6), lambda k: (0, 0)),
            pl.BlockSpec((TAB, D6), lambda k: (0, 0)),
            pl.BlockSpec((TAB, D6), lambda k: (0, 0)),
            pl.BlockSpec((DOUT, D6), lambda k: (0, k)),
            pl.BlockSpec((1, DOUT), lambda k: (0, 0)),
        ],
        out_specs=pl.BlockSpec((TAB, DOUT), lambda k: (k, 0)),
        out_shape=jax.ShapeDtypeStruct((NSLOT * TAB, DOUT), jnp.bfloat16),
    )(x_emb, y_emb, h_emb, w_emb, lin_w, bias)


def _sc_lookup_body(lids_hbm, pos_hbm, proj_hbm, box_hbm, out_hbm,
                    lids_v, pos_v, idx6_v, rows6_a, rows6_b,
                    rowsb_a, rowsb_b, out_a, out_b, sem_a, sem_b,
                    osem_a, osem_b):
    wid = lax.axis_index("s") * NC + lax.axis_index("c")
    base = wid * TPW

    pltpu.sync_copy(lids_hbm.at[pl.ds(base * 4, TPW * 4)], lids_v)
    pltpu.sync_copy(pos_hbm.at[pl.ds(base, TPW)], pos_v)

    lane4 = lax.iota(jnp.int32, LANES) * 4
    for j in range(TPW // LANES):
        t = j * LANES
        ch = t // CH
        off = t - ch * CH
        c0 = plsc.load_gather(lids_v, [lane4 + (t * 4 + 0)])
        c1 = plsc.load_gather(lids_v, [lane4 + (t * 4 + 1)])
        c2 = plsc.load_gather(lids_v, [lane4 + (t * 4 + 2)])
        c3 = plsc.load_gather(lids_v, [lane4 + (t * 4 + 3)])
        slots = (
            c0,
            c1 + TAB,
            c2 + 2 * TAB,
            c3 + 3 * TAB,
            (c3 - c1) + 4 * TAB,
            (c2 - c0) + 5 * TAB,
        )
        for k in range(NSLOT):
            idx6_v[pl.ds(ch * (NSLOT * CH) + k * CH + off, LANES)] = slots[k]

    rows6 = (rows6_a, rows6_b)
    rowsb = (rowsb_a, rowsb_b)
    outs = (out_a, out_b)
    sems = (sem_a, sem_b)
    osems = (osem_a, osem_b)
    third = NSLOT * CH // 3

    def fire(ch):
        par = ch % 2
        cbase = ch * (NSLOT * CH)
        hs = tuple(
            pltpu.async_copy(
                proj_hbm.at[idx6_v.at[pl.ds(cbase + p * third, third)]],
                rows6[par].at[pl.ds(p * third, third)], sems[par])
            for p in range(3))
        return hs + (
            pltpu.async_copy(
                box_hbm.at[pos_v.at[pl.ds(ch * CH, CH)]],
                rowsb[par], sems[par]),
        )

    handles = fire(0)
    ohandles = (None, None)
    for ch in range(NCH):
        par = ch % 2
        nxt = fire(ch + 1) if ch + 1 < NCH else None
        for h in handles:
            h.wait()
        if ohandles[par] is not None:
            ohandles[par].wait()
        r6, rb, ov = rows6[par], rowsb[par], outs[par]

        @plsc.parallel_loop(0, CH, unroll=4)
        def acc_body(i):
            for g in range(DOUT // 32):
                gsl = pl.ds(g * 32, 32)
                s = r6[0 * CH + i, gsl]
                for k in range(1, NSLOT):
                    s = s + r6[k * CH + i, gsl]
                lo, hi = plsc.unpack(s, format=plsc.PackFormat.INTERLEAVED)
                lo = lo + rb[i, pl.ds(g * 32, LANES)]
                hi = hi + rb[i, pl.ds(g * 32 + LANES, LANES)]
                ov[i, pl.ds(g * 32, LANES)] = lo
                ov[i, pl.ds(g * 32 + LANES, LANES)] = hi
        oh = pltpu.async_copy(ov, out_hbm.at[pl.ds(base + ch * CH, CH)],
                              osems[par])
        ohandles = tuple(oh if q == par else ohandles[q] for q in range(2))
        handles = nxt
    for q in range(2):
        if ohandles[q] is not None:
            ohandles[q].wait()


def _sc_lookup(lids_flat, pos_flat, proj, box):
    mesh = plsc.VectorSubcoreMesh(core_axis_name="c", subcore_axis_name="s")
    f = functools.partial(
        pl.kernel,
        mesh=mesh,
        out_type=jax.ShapeDtypeStruct((NTOK, DOUT), jnp.float32),
        scratch_types=[
            pltpu.VMEM((TPW * 4,), jnp.int32),
            pltpu.VMEM((TPW,), jnp.int32),
            pltpu.VMEM((NSLOT * TPW,), jnp.int32),
            pltpu.VMEM((NSLOT * CH, DOUT), jnp.bfloat16),
            pltpu.VMEM((NSLOT * CH, DOUT), jnp.bfloat16),
            pltpu.VMEM((CH, DOUT), jnp.float32),
            pltpu.VMEM((CH, DOUT), jnp.float32),
            pltpu.VMEM((CH, DOUT), jnp.float32),
            pltpu.VMEM((CH, DOUT), jnp.float32),
            pltpu.SemaphoreType.DMA,
            pltpu.SemaphoreType.DMA,
            pltpu.SemaphoreType.DMA,
            pltpu.SemaphoreType.DMA,
        ],
        compiler_params=pltpu.CompilerParams(use_tc_tiling_on_sc=False,
                                             needs_layout_passes=False),
    )(_sc_lookup_body)
    return f(lids_flat, pos_flat, proj, box)


def kernel(layout_ids, position_ids, x_emb, y_emb, h_emb, w_emb, box_emb,
           lin_w, lin_b):
    proj = _tc_project(x_emb, y_emb, h_emb, w_emb, lin_w,
                       lin_b.reshape(1, DOUT))
    lids_flat = layout_ids.astype(jnp.int32).reshape(-1)
    pos_flat = position_ids.astype(jnp.int32).reshape(-1)
    out = _sc_lookup(lids_flat, pos_flat, proj, box_emb)
    return out.reshape(B, S, DOUT)


# speedup vs baseline: 1.0475x; 1.0475x over previous
import functools

import jax
import jax.numpy as jnp
from jax import lax
from jax.experimental import pallas as pl
from jax.experimental.pallas import tpu as pltpu
from jax.experimental.pallas import tpu_sc as plsc

B, S = 4, 2048
NTOK = B * S
D6 = 128
DOUT = 192
NSLOT = 6
TAB = 1024

_info = plsc.get_sparse_core_info()
NC = _info.num_cores
NS = _info.num_subcores
NW = NC * NS
TPW = NTOK // NW
CH = 64
NCH = TPW // CH
LANES = 16


def _tc_project_body(x_ref, y_ref, h_ref, w_ref, lw_ref, b_ref, out_ref):
    k = pl.program_id(0)
    emb = jnp.where(
        (k == 0) | (k == 2), x_ref[...],
        jnp.where((k == 1) | (k == 3), y_ref[...],
                  jnp.where(k == 4, h_ref[...], w_ref[...])))
    acc = lax.dot_general(emb, lw_ref[...], (((1,), (1,)), ((), ())),
                          preferred_element_type=jnp.float32)
    acc = acc + jnp.where(k == 0, b_ref[...], 0.0)
    cols = lax.broadcasted_iota(jnp.int32, (DOUT, DOUT), 1)
    rows = lax.broadcasted_iota(jnp.int32, (DOUT, DOUT), 0)
    src = (cols // 32) * 32 + (cols % 2) * 16 + (cols % 32) // 2
    perm = (rows == src).astype(jnp.float32)
    out_ref[...] = lax.dot_general(
        acc, perm, (((1,), (0,)), ((), ())),
        preferred_element_type=jnp.float32).astype(jnp.bfloat16)


def _tc_project(x_emb, y_emb, h_emb, w_emb, lin_w, bias):
    return pl.pallas_call(
        _tc_project_body,
        grid=(NSLOT,),
        in_specs=[
            pl.BlockSpec((TAB, D6), lambda k: (0, 0)),
            pl.BlockSpec((TAB, D6), lambda k: (0, 0)),
            pl.BlockSpec((TAB, D6), lambda k: (0, 0)),
            pl.BlockSpec((TAB, D6), lambda k: (0, 0)),
            pl.BlockSpec((DOUT, D6), lambda k: (0, k)),
            pl.BlockSpec((1, DOUT), lambda k: (0, 0)),
        ],
        out_specs=pl.BlockSpec((TAB, DOUT), lambda k: (k, 0)),
        out_shape=jax.ShapeDtypeStruct((NSLOT * TAB, DOUT), jnp.bfloat16),
    )(x_emb, y_emb, h_emb, w_emb, lin_w, bias)


def _sc_lookup_body(lids_hbm, pos_hbm, proj_hbm, box_hbm, out_hbm,
                    lids_v, pos_v, idx6_v, rows6_a, rows6_b,
                    rowsb_a, rowsb_b, out_a, out_b, sem_a, sem_b,
                    osem_a, osem_b):
    wid = lax.axis_index("s") * NC + lax.axis_index("c")
    base = wid * TPW

    for c in range(4):
        pltpu.sync_copy(lids_hbm.at[pl.ds(c * NTOK + base, TPW)],
                        lids_v.at[pl.ds(c * TPW, TPW)])
    pltpu.sync_copy(pos_hbm.at[pl.ds(base, TPW)], pos_v)

    for j in range(TPW // LANES):
        t = j * LANES
        ch = t // CH
        off = t - ch * CH
        c0 = lids_v[pl.ds(0 * TPW + t, LANES)]
        c1 = lids_v[pl.ds(1 * TPW + t, LANES)]
        c2 = lids_v[pl.ds(2 * TPW + t, LANES)]
        c3 = lids_v[pl.ds(3 * TPW + t, LANES)]
        slots = (
            c0,
            c1 + TAB,
            c2 + 2 * TAB,
            c3 + 3 * TAB,
            (c3 - c1) + 4 * TAB,
            (c2 - c0) + 5 * TAB,
        )
        for k in range(NSLOT):
            idx6_v[pl.ds(ch * (NSLOT * CH) + k * CH + off, LANES)] = slots[k]

    rows6 = (rows6_a, rows6_b)
    rowsb = (rowsb_a, rowsb_b)
    outs = (out_a, out_b)
    sems = (sem_a, sem_b)
    osems = (osem_a, osem_b)
    third = NSLOT * CH // 3

    def fire(ch):
        par = ch % 2
        cbase = ch * (NSLOT * CH)
        hs = tuple(
            pltpu.async_copy(
                proj_hbm.at[idx6_v.at[pl.ds(cbase + p * third, third)]],
                rows6[par].at[pl.ds(p * third, third)], sems[par])
            for p in range(3))
        return hs + (
            pltpu.async_copy(
                box_hbm.at[pos_v.at[pl.ds(ch * CH, CH)]],
                rowsb[par], sems[par]),
        )

    handles = fire(0)
    ohandles = (None, None)
    for ch in range(NCH):
        par = ch % 2
        nxt = fire(ch + 1) if ch + 1 < NCH else None
        for h in handles:
            h.wait()
        if ohandles[par] is not None:
            ohandles[par].wait()
        r6, rb, ov = rows6[par], rowsb[par], outs[par]

        @plsc.parallel_loop(0, CH, unroll=4)
        def acc_body(i):
            for g in range(DOUT // 32):
                gsl = pl.ds(g * 32, 32)
                s = r6[0 * CH + i, gsl]
                for k in range(1, NSLOT):
                    s = s + r6[k * CH + i, gsl]
                lo, hi = plsc.unpack(s, format=plsc.PackFormat.INTERLEAVED)
                lo = lo + rb[i, pl.ds(g * 32, LANES)]
                hi = hi + rb[i, pl.ds(g * 32 + LANES, LANES)]
                ov[i, pl.ds(g * 32, LANES)] = lo
                ov[i, pl.ds(g * 32 + LANES, LANES)] = hi
        oh = pltpu.async_copy(ov, out_hbm.at[pl.ds(base + ch * CH, CH)],
                              osems[par])
        ohandles = tuple(oh if q == par else ohandles[q] for q in range(2))
        handles = nxt
    for q in range(2):
        if ohandles[q] is not None:
            ohandles[q].wait()


def _sc_lookup(lids_flat, pos_flat, proj, box):
    mesh = plsc.VectorSubcoreMesh(core_axis_name="c", subcore_axis_name="s")
    f = functools.partial(
        pl.kernel,
        mesh=mesh,
        out_type=jax.ShapeDtypeStruct((NTOK, DOUT), jnp.float32),
        scratch_types=[
            pltpu.VMEM((TPW * 4,), jnp.int32),
            pltpu.VMEM((TPW,), jnp.int32),
            pltpu.VMEM((NSLOT * TPW,), jnp.int32),
            pltpu.VMEM((NSLOT * CH, DOUT), jnp.bfloat16),
            pltpu.VMEM((NSLOT * CH, DOUT), jnp.bfloat16),
            pltpu.VMEM((CH, DOUT), jnp.float32),
            pltpu.VMEM((CH, DOUT), jnp.float32),
            pltpu.VMEM((CH, DOUT), jnp.float32),
            pltpu.VMEM((CH, DOUT), jnp.float32),
            pltpu.SemaphoreType.DMA,
            pltpu.SemaphoreType.DMA,
            pltpu.SemaphoreType.DMA,
            pltpu.SemaphoreType.DMA,
        ],
        compiler_params=pltpu.CompilerParams(use_tc_tiling_on_sc=False,
                                             needs_layout_passes=False),
    )(_sc_lookup_body)
    return f(lids_flat, pos_flat, proj, box)


def kernel(layout_ids, position_ids, x_emb, y_emb, h_emb, w_emb, box_emb,
           lin_w, lin_b):
    proj = _tc_project(x_emb, y_emb, h_emb, w_emb, lin_w,
                       lin_b.reshape(1, DOUT))
    lids_flat = jnp.maximum(
        layout_ids.astype(jnp.int32).transpose(2, 0, 1), 0).reshape(-1)
    pos_flat = position_ids.astype(jnp.int32).reshape(-1)
    out = _sc_lookup(lids_flat, pos_flat, proj, box_emb)
    return out.reshape(B, S, DOUT)
